# baseline (speedup 1.0000x reference)
import functools

import jax
import jax.numpy as jnp
from jax import lax
from jax.experimental import pallas as pl
from jax.experimental.pallas import tpu as pltpu
from jax.experimental.pallas import tpu_sc as plsc

N = 10000
E = 320000
F_IN = 128
H = 128
C = 64
H2 = H // 2
C2 = C // 2

ROWS_PER_SUB = N // 16
KG_H = 500
KG_C = 1000
DEG_KG = 1000
DEG_STEPS = E // (32 * DEG_KG)
DEG_W = 16

_BLK = 2000
_GRID = N // _BLK


def _sc_mesh():
    return plsc.VectorSubcoreMesh(core_axis_name="c", subcore_axis_name="s")


_SC_PARAMS = pltpu.CompilerParams(use_tc_tiling_on_sc=False)


@functools.partial(
    pl.kernel,
    out_type=jax.ShapeDtypeStruct((2, N, DEG_W), jnp.float32),
    mesh=_sc_mesh(),
    compiler_params=_SC_PARAMS,
    scratch_types=[
        pltpu.VMEM((DEG_STEPS, DEG_KG), jnp.int32),
        pltpu.VMEM((DEG_KG, DEG_W), jnp.float32),
        pltpu.VMEM_SHARED((N, DEG_W), jnp.float32),
    ],
)
def _deg_kernel(row_hbm, ones_hbm, zeros_hbm, out_hbm, row_v, ones_v, acc):
    c = lax.axis_index("c")
    s = lax.axis_index("s")
    start = (c * 16 + s) * DEG_STEPS
    pltpu.sync_copy(row_hbm.at[pl.ds(start, DEG_STEPS)], row_v)
    pltpu.sync_copy(ones_hbm, ones_v)
    pltpu.sync_copy(zeros_hbm, acc.at[pl.ds(s * ROWS_PER_SUB, ROWS_PER_SUB)])
    plsc.subcore_barrier()

    @pl.loop(0, DEG_STEPS)
    def _(j):
        pltpu.sync_copy(ones_v, acc.at[row_v.at[j]], add=True)

    plsc.subcore_barrier()
    pltpu.sync_copy(
        acc.at[pl.ds(s * ROWS_PER_SUB, ROWS_PER_SUB)],
        out_hbm.at[c, pl.ds(s * ROWS_PER_SUB, ROWS_PER_SUB)],
    )


def _make_spmm(F2, KG):
    nsteps = E // (16 * KG)

    @functools.partial(
        pl.kernel,
        out_type=jax.ShapeDtypeStruct((2, N, F2), jnp.float32),
        mesh=_sc_mesh(),
        compiler_params=_SC_PARAMS,
        scratch_types=[
            pltpu.VMEM((2, KG), jnp.int32),
            pltpu.VMEM((2, KG), jnp.int32),
            pltpu.VMEM((KG, F2), jnp.float32),
            pltpu.VMEM((KG, F2), jnp.float32),
            pltpu.SemaphoreType.DMA,
            pltpu.SemaphoreType.DMA,
            pltpu.SemaphoreType.DMA,
            pltpu.SemaphoreType.DMA,
            pltpu.VMEM_SHARED((N, F2), jnp.float32),
        ],
    )
    def spmm(s_hbm, idx_hbm, zeros_hbm, out_hbm,
             iv0, iv1, gb0, gb1, gsem0, gsem1, isem0, isem1, acc):
        c = lax.axis_index("c")
        s = lax.axis_index("s")
        base = s * nsteps
        pltpu.sync_copy(zeros_hbm, acc.at[pl.ds(s * ROWS_PER_SUB, ROWS_PER_SUB)])
        plsc.subcore_barrier()
        src = s_hbm.at[c]

        def wait_g(gb, gsem):
            pltpu.make_async_copy(src.at[pl.ds(0, KG)], gb, gsem).wait()

        def wait_i(iv, isem):
            pltpu.make_async_copy(idx_hbm.at[0], iv, isem).wait()

        pltpu.sync_copy(idx_hbm.at[base], iv0)
        pltpu.async_copy(src.at[iv0.at[1]], gb0, gsem0)
        pltpu.async_copy(idx_hbm.at[base + 1], iv1, isem1)

        @pl.loop(0, nsteps, step=2)
        def _(j):
            wait_g(gb0, gsem0)
            wait_i(iv1, isem1)
            pltpu.async_copy(src.at[iv1.at[1]], gb1, gsem1)
            pltpu.sync_copy(gb0, acc.at[iv0.at[0]], add=True)

            @pl.when(j + 2 < nsteps)
            def _():
                pltpu.async_copy(idx_hbm.at[base + j + 2], iv0, isem0)

            wait_g(gb1, gsem1)

            @pl.when(j + 2 < nsteps)
            def _():
                wait_i(iv0, isem0)
                pltpu.async_copy(src.at[iv0.at[1]], gb0, gsem0)

            pltpu.sync_copy(gb1, acc.at[iv1.at[0]], add=True)

            @pl.when(j + 3 < nsteps)
            def _():
                pltpu.async_copy(idx_hbm.at[base + j + 3], iv1, isem1)

        plsc.subcore_barrier()
        pltpu.sync_copy(
            acc.at[pl.ds(s * ROWS_PER_SUB, ROWS_PER_SUB)],
            out_hbm.at[c, pl.ds(s * ROWS_PER_SUB, ROWS_PER_SUB)],
        )

    return spmm


_spmm_h = _make_spmm(H2, KG_H)
_spmm_c = _make_spmm(C2, KG_C)


def _mm1_body(deg_ref, x_ref, w_ref, o_ref, dinv_ref):
    deg = deg_ref[0] + deg_ref[1] + 1.0
    dinv = lax.rsqrt(deg)
    s1 = jnp.dot(x_ref[...], w_ref[...], preferred_element_type=jnp.float32)
    s1p = s1 * dinv
    o_ref[0] = s1p[:, :H2]
    o_ref[1] = s1p[:, H2:]
    dinv_ref[...] = dinv


_mm1 = pl.pallas_call(
    _mm1_body,
    grid=(_GRID,),
    in_specs=[
        pl.BlockSpec((2, _BLK, 1), lambda i: (0, i, 0)),
        pl.BlockSpec((_BLK, F_IN), lambda i: (i, 0)),
        pl.BlockSpec((F_IN, H), lambda i: (0, 0)),
    ],
    out_specs=[
        pl.BlockSpec((2, _BLK, H2), lambda i: (0, i, 0)),
        pl.BlockSpec((_BLK, 1), lambda i: (i, 0)),
    ],
    out_shape=[
        jax.ShapeDtypeStruct((2, N, H2), jnp.float32),
        jax.ShapeDtypeStruct((N, 1), jnp.float32),
    ],
)


def _mid_body(dinv_ref, p_ref, s1p_ref, b1_ref, w2_ref, o_ref):
    dinv = dinv_ref[...]
    tot = jnp.concatenate(
        [p_ref[0] + s1p_ref[0], p_ref[1] + s1p_ref[1]], axis=-1)
    h = tot * dinv + b1_ref[...]
    h = jnp.maximum(h, 0.0)
    s2 = jnp.dot(h, w2_ref[...], preferred_element_type=jnp.float32)
    s2p = s2 * dinv
    o_ref[0] = s2p[:, :C2]
    o_ref[1] = s2p[:, C2:]


_mid = pl.pallas_call(
    _mid_body,
    grid=(_GRID,),
    in_specs=[
        pl.BlockSpec((_BLK, 1), lambda i: (i, 0)),
        pl.BlockSpec((2, _BLK, H2), lambda i: (0, i, 0)),
        pl.BlockSpec((2, _BLK, H2), lambda i: (0, i, 0)),
        pl.BlockSpec((1, H), lambda i: (0, 0)),
        pl.BlockSpec((H, C), lambda i: (0, 0)),
    ],
    out_specs=pl.BlockSpec((2, _BLK, C2), lambda i: (0, i, 0)),
    out_shape=jax.ShapeDtypeStruct((2, N, C2), jnp.float32),
)


def _final_body(dinv_ref, q_ref, s2p_ref, b2_ref, o_ref):
    dinv = dinv_ref[...]
    tot = jnp.concatenate(
        [q_ref[0] + s2p_ref[0], q_ref[1] + s2p_ref[1]], axis=-1)
    o_ref[...] = tot * dinv + b2_ref[...]


_final = pl.pallas_call(
    _final_body,
    grid=(_GRID,),
    in_specs=[
        pl.BlockSpec((_BLK, 1), lambda i: (i, 0)),
        pl.BlockSpec((2, _BLK, C2), lambda i: (0, i, 0)),
        pl.BlockSpec((2, _BLK, C2), lambda i: (0, i, 0)),
        pl.BlockSpec((1, C), lambda i: (0, 0)),
    ],
    out_specs=pl.BlockSpec((_BLK, C), lambda i: (i, 0)),
    out_shape=jax.ShapeDtypeStruct((N, C), jnp.float32),
)


def kernel(x, edge_index, W1, b1, W2, b2):
    row = edge_index[0]
    col = edge_index[1]
    row_deg = row.reshape(32 * DEG_STEPS, DEG_KG)
    idx_h = jnp.stack([row.reshape(-1, KG_H), col.reshape(-1, KG_H)], axis=1)
    idx_c = jnp.stack([row.reshape(-1, KG_C), col.reshape(-1, KG_C)], axis=1)

    ones_deg = jnp.ones((DEG_KG, DEG_W), jnp.float32)
    zeros_deg = jnp.zeros((ROWS_PER_SUB, DEG_W), jnp.float32)
    zeros_h = jnp.zeros((ROWS_PER_SUB, H2), jnp.float32)
    zeros_c = jnp.zeros((ROWS_PER_SUB, C2), jnp.float32)

    degp = _deg_kernel(row_deg, ones_deg, zeros_deg)
    s1p, dinv = _mm1(degp[:, :, :1], x, W1)
    p = _spmm_h(s1p, idx_h, zeros_h)
    s2p = _mid(dinv, p, s1p, b1.reshape(1, H), W2)
    q = _spmm_c(s2p, idx_c, zeros_c)
    out = _final(dinv, q, s2p, b2.reshape(1, C))
    return out


# speedup vs baseline: 1.0073x; 1.0073x over previous
import functools

import jax
import jax.numpy as jnp
from jax import lax
from jax.experimental import pallas as pl
from jax.experimental.pallas import tpu as pltpu
from jax.experimental.pallas import tpu_sc as plsc

N = 10000
E = 320000
F_IN = 128
H = 128
C = 64
H2 = H // 2
C2 = C // 2

ROWS_PER_SUB = N // 16
KG_H = 625
KG_C = 1000
DEG_KG = 1000
DEG_STEPS = E // (32 * DEG_KG)
DEG_W = 16

_BLK = 2000
_GRID = N // _BLK


def _sc_mesh():
    return plsc.VectorSubcoreMesh(core_axis_name="c", subcore_axis_name="s")


_SC_PARAMS = pltpu.CompilerParams(use_tc_tiling_on_sc=False)


@functools.partial(
    pl.kernel,
    out_type=jax.ShapeDtypeStruct((2, N, DEG_W), jnp.float32),
    mesh=_sc_mesh(),
    compiler_params=_SC_PARAMS,
    scratch_types=[
        pltpu.VMEM((DEG_STEPS, DEG_KG), jnp.int32),
        pltpu.VMEM((DEG_KG, DEG_W), jnp.float32),
        pltpu.VMEM_SHARED((N, DEG_W), jnp.float32),
    ],
)
def _deg_kernel(row_hbm, ones_hbm, zeros_hbm, out_hbm, row_v, ones_v, acc):
    c = lax.axis_index("c")
    s = lax.axis_index("s")
    start = (c * 16 + s) * DEG_STEPS
    pltpu.sync_copy(row_hbm.at[pl.ds(start, DEG_STEPS)], row_v)
    pltpu.sync_copy(ones_hbm, ones_v)
    pltpu.sync_copy(zeros_hbm, acc.at[pl.ds(s * ROWS_PER_SUB, ROWS_PER_SUB)])
    plsc.subcore_barrier()

    @pl.loop(0, DEG_STEPS)
    def _(j):
        pltpu.sync_copy(ones_v, acc.at[row_v.at[j]], add=True)

    plsc.subcore_barrier()
    pltpu.sync_copy(
        acc.at[pl.ds(s * ROWS_PER_SUB, ROWS_PER_SUB)],
        out_hbm.at[c, pl.ds(s * ROWS_PER_SUB, ROWS_PER_SUB)],
    )


def _make_spmm(F2, KG):
    nsteps = E // (16 * KG)

    @functools.partial(
        pl.kernel,
        out_type=jax.ShapeDtypeStruct((2, N, F2), jnp.float32),
        mesh=_sc_mesh(),
        compiler_params=_SC_PARAMS,
        scratch_types=[
            pltpu.VMEM((2, KG), jnp.int32),
            pltpu.VMEM((2, KG), jnp.int32),
            pltpu.VMEM((KG, F2), jnp.float32),
            pltpu.VMEM((KG, F2), jnp.float32),
            pltpu.SemaphoreType.DMA,
            pltpu.SemaphoreType.DMA,
            pltpu.SemaphoreType.DMA,
            pltpu.SemaphoreType.DMA,
            pltpu.VMEM_SHARED((N, F2), jnp.float32),
        ],
    )
    def spmm(s_hbm, idx_hbm, zeros_hbm, out_hbm,
             iv0, iv1, gb0, gb1, gsem0, gsem1, isem0, isem1, acc):
        c = lax.axis_index("c")
        s = lax.axis_index("s")
        base = s * nsteps
        pltpu.sync_copy(zeros_hbm, acc.at[pl.ds(s * ROWS_PER_SUB, ROWS_PER_SUB)])
        plsc.subcore_barrier()
        src = s_hbm.at[c]

        def wait_g(gb, gsem):
            pltpu.make_async_copy(src.at[pl.ds(0, KG)], gb, gsem).wait()

        def wait_i(iv, isem):
            pltpu.make_async_copy(idx_hbm.at[0], iv, isem).wait()

        pltpu.sync_copy(idx_hbm.at[base], iv0)
        pltpu.async_copy(src.at[iv0.at[1]], gb0, gsem0)
        pltpu.async_copy(idx_hbm.at[base + 1], iv1, isem1)

        @pl.loop(0, nsteps, step=2)
        def _(j):
            wait_g(gb0, gsem0)
            wait_i(iv1, isem1)
            pltpu.async_copy(src.at[iv1.at[1]], gb1, gsem1)
            pltpu.sync_copy(gb0, acc.at[iv0.at[0]], add=True)

            @pl.when(j + 2 < nsteps)
            def _():
                pltpu.async_copy(idx_hbm.at[base + j + 2], iv0, isem0)

            wait_g(gb1, gsem1)

            @pl.when(j + 2 < nsteps)
            def _():
                wait_i(iv0, isem0)
                pltpu.async_copy(src.at[iv0.at[1]], gb0, gsem0)

            pltpu.sync_copy(gb1, acc.at[iv1.at[0]], add=True)

            @pl.when(j + 3 < nsteps)
            def _():
                pltpu.async_copy(idx_hbm.at[base + j + 3], iv1, isem1)

        plsc.subcore_barrier()
        pltpu.sync_copy(
            acc.at[pl.ds(s * ROWS_PER_SUB, ROWS_PER_SUB)],
            out_hbm.at[c, pl.ds(s * ROWS_PER_SUB, ROWS_PER_SUB)],
        )

    return spmm


_spmm_h = _make_spmm(H2, KG_H)
_spmm_c = _make_spmm(C2, KG_C)


def _mm1_body(deg_ref, x_ref, w_ref, o_ref, dinv_ref):
    deg = deg_ref[0] + deg_ref[1] + 1.0
    dinv = lax.rsqrt(deg)
    s1 = jnp.dot(x_ref[...], w_ref[...], preferred_element_type=jnp.float32)
    s1p = s1 * dinv
    o_ref[0] = s1p[:, :H2]
    o_ref[1] = s1p[:, H2:]
    dinv_ref[...] = dinv


_mm1 = pl.pallas_call(
    _mm1_body,
    grid=(_GRID,),
    in_specs=[
        pl.BlockSpec((2, _BLK, 1), lambda i: (0, i, 0)),
        pl.BlockSpec((_BLK, F_IN), lambda i: (i, 0)),
        pl.BlockSpec((F_IN, H), lambda i: (0, 0)),
    ],
    out_specs=[
        pl.BlockSpec((2, _BLK, H2), lambda i: (0, i, 0)),
        pl.BlockSpec((_BLK, 1), lambda i: (i, 0)),
    ],
    out_shape=[
        jax.ShapeDtypeStruct((2, N, H2), jnp.float32),
        jax.ShapeDtypeStruct((N, 1), jnp.float32),
    ],
)


def _mid_body(dinv_ref, p_ref, s1p_ref, b1_ref, w2_ref, o_ref):
    dinv = dinv_ref[...]
    tot = jnp.concatenate(
        [p_ref[0] + s1p_ref[0], p_ref[1] + s1p_ref[1]], axis=-1)
    h = tot * dinv + b1_ref[...]
    h = jnp.maximum(h, 0.0)
    s2 = jnp.dot(h, w2_ref[...], preferred_element_type=jnp.float32)
    s2p = s2 * dinv
    o_ref[0] = s2p[:, :C2]
    o_ref[1] = s2p[:, C2:]


_mid = pl.pallas_call(
    _mid_body,
    grid=(_GRID,),
    in_specs=[
        pl.BlockSpec((_BLK, 1), lambda i: (i, 0)),
        pl.BlockSpec((2, _BLK, H2), lambda i: (0, i, 0)),
        pl.BlockSpec((2, _BLK, H2), lambda i: (0, i, 0)),
        pl.BlockSpec((1, H), lambda i: (0, 0)),
        pl.BlockSpec((H, C), lambda i: (0, 0)),
    ],
    out_specs=pl.BlockSpec((2, _BLK, C2), lambda i: (0, i, 0)),
    out_shape=jax.ShapeDtypeStruct((2, N, C2), jnp.float32),
)


def _final_body(dinv_ref, q_ref, s2p_ref, b2_ref, o_ref):
    dinv = dinv_ref[...]
    tot = jnp.concatenate(
        [q_ref[0] + s2p_ref[0], q_ref[1] + s2p_ref[1]], axis=-1)
    o_ref[...] = tot * dinv + b2_ref[...]


_final = pl.pallas_call(
    _final_body,
    grid=(_GRID,),
    in_specs=[
        pl.BlockSpec((_BLK, 1), lambda i: (i, 0)),
        pl.BlockSpec((2, _BLK, C2), lambda i: (0, i, 0)),
        pl.BlockSpec((2, _BLK, C2), lambda i: (0, i, 0)),
        pl.BlockSpec((1, C), lambda i: (0, 0)),
    ],
    out_specs=pl.BlockSpec((_BLK, C), lambda i: (i, 0)),
    out_shape=jax.ShapeDtypeStruct((N, C), jnp.float32),
)


def kernel(x, edge_index, W1, b1, W2, b2):
    row = edge_index[0]
    col = edge_index[1]
    row_deg = row.reshape(32 * DEG_STEPS, DEG_KG)
    idx_h = jnp.stack([row.reshape(-1, KG_H), col.reshape(-1, KG_H)], axis=1)
    idx_c = jnp.stack([row.reshape(-1, KG_C), col.reshape(-1, KG_C)], axis=1)

    ones_deg = jnp.ones((DEG_KG, DEG_W), jnp.float32)
    zeros_deg = jnp.zeros((ROWS_PER_SUB, DEG_W), jnp.float32)
    zeros_h = jnp.zeros((ROWS_PER_SUB, H2), jnp.float32)
    zeros_c = jnp.zeros((ROWS_PER_SUB, C2), jnp.float32)

    degp = _deg_kernel(row_deg, ones_deg, zeros_deg)
    s1p, dinv = _mm1(degp[:, :, :1], x, W1)
    p = _spmm_h(s1p, idx_h, zeros_h)
    s2p = _mid(dinv, p, s1p, b1.reshape(1, H), W2)
    q = _spmm_c(s2p, idx_c, zeros_c)
    out = _final(dinv, q, s2p, b2.reshape(1, C))
    return out
